# baseline (speedup 1.0000x reference)
import jax
import jax.numpy as jnp
from jax import lax
from jax.experimental import pallas as pl
from jax.experimental.pallas import tpu as pltpu

B, S, D, E = 2, 2048, 1024, 8
RC = 64
NRC = S // RC
NK = D // 128
EPS2 = 1e-24


def _fused_kernel(s_ref, t_ref, t_any, o_any, acc_ref, rs_ref, sem0, sem1):
    e = pl.program_id(0)
    for b in range(B):
        @pl.when(e == 0)
        def _():
            for rc in range(NRC):
                r0 = rc * RC
                sn_acc = jnp.zeros((RC, 128), jnp.float32)
                for k in range(NK):
                    sfk = s_ref[b, r0:r0 + RC, k * 128:(k + 1) * 128]
                    sn_acc += sfk * sfk
                sn2 = jnp.sum(sn_acc, axis=1, keepdims=True)
                rs_ref[b, r0:r0 + RC, :] = lax.rsqrt(jnp.maximum(sn2, EPS2))

        part = jnp.zeros((1, 1), jnp.float32)
        for rc in range(NRC):
            r0 = rc * RC
            dot_acc = jnp.zeros((RC, 128), jnp.float32)
            tn_acc = jnp.zeros((RC, 128), jnp.float32)
            for k in range(NK):
                sfk = s_ref[b, r0:r0 + RC, k * 128:(k + 1) * 128]
                tfk = t_ref[0, b, r0:r0 + RC, k * 128:(k + 1) * 128]
                dot_acc += sfk * tfk
                tn_acc += tfk * tfk
            dot = jnp.sum(dot_acc, axis=1, keepdims=True)
            tn2 = jnp.sum(tn_acc, axis=1, keepdims=True)
            rt = lax.rsqrt(jnp.maximum(tn2, EPS2))
            w = dot * rt * rs_ref[b, r0:r0 + RC, :]
            part += jnp.sum(w, axis=0, keepdims=True)
        acc_ref[b, pl.ds(e, 1), :] = part

    @pl.when(e == E - 1)
    def _():
        i0 = jnp.argmax(acc_ref[0][:, 0], axis=0)
        i1 = jnp.argmax(acc_ref[1][:, 0], axis=0)
        cp0 = pltpu.make_async_copy(t_any.at[i0, 0], o_any.at[0], sem0)
        cp1 = pltpu.make_async_copy(t_any.at[i1, 1], o_any.at[1], sem1)
        cp0.start()
        cp1.start()
        cp0.wait()
        cp1.wait()


@jax.jit
def kernel(student_features, teacher_features):
    return pl.pallas_call(
        _fused_kernel,
        grid=(E,),
        in_specs=[
            pl.BlockSpec((B, S, D), lambda e: (0, 0, 0)),
            pl.BlockSpec((1, B, S, D), lambda e: (e, 0, 0, 0)),
            pl.BlockSpec(memory_space=pl.ANY),
        ],
        out_specs=pl.BlockSpec(memory_space=pl.ANY),
        out_shape=jax.ShapeDtypeStruct((B, S, D), jnp.float32),
        scratch_shapes=[
            pltpu.VMEM((B, E, 1), jnp.float32),
            pltpu.VMEM((B, S, 1), jnp.float32),
            pltpu.SemaphoreType.DMA,
            pltpu.SemaphoreType.DMA,
        ],
        compiler_params=pltpu.CompilerParams(
            dimension_semantics=("arbitrary",),
        ),
    )(student_features, teacher_features, teacher_features)


# speedup vs baseline: 8.9103x; 8.9103x over previous
import functools

import jax
import jax.numpy as jnp
from jax import lax
from jax.experimental import pallas as pl
from jax.experimental.pallas import tpu as pltpu

B, S, D, E = 2, 2048, 1024, 8
S_BLK = 2048
NS = S // S_BLK
C_BLK = 512
NCB = S // C_BLK
RC = 64
NRC = S_BLK // RC
NK = D // 128
EPS2 = 1e-24


def _sims_kernel(s_ref, t_ref, idx_ref, acc_ref, rs_ref):
    s = pl.program_id(0)
    e = pl.program_id(1)
    for b in range(B):
        @pl.when(e == 0)
        def _():
            for rc in range(NRC):
                r0 = rc * RC
                sn_acc = jnp.zeros((RC, 128), jnp.float32)
                for k in range(NK):
                    sfk = s_ref[b, r0:r0 + RC, k * 128:(k + 1) * 128]
                    sn_acc += sfk * sfk
                sn2 = jnp.sum(sn_acc, axis=1, keepdims=True)
                rs_ref[b, r0:r0 + RC, :] = lax.rsqrt(jnp.maximum(sn2, EPS2))

        part = jnp.zeros((1, 1), jnp.float32)
        for rc in range(NRC):
            r0 = rc * RC
            dot_acc = jnp.zeros((RC, 128), jnp.float32)
            tn_acc = jnp.zeros((RC, 128), jnp.float32)
            for k in range(NK):
                sfk = s_ref[b, r0:r0 + RC, k * 128:(k + 1) * 128]
                tfk = t_ref[0, b, r0:r0 + RC, k * 128:(k + 1) * 128]
                dot_acc += sfk * tfk
                tn_acc += tfk * tfk
            dot = jnp.sum(dot_acc, axis=1, keepdims=True)
            tn2 = jnp.sum(tn_acc, axis=1, keepdims=True)
            rt = lax.rsqrt(jnp.maximum(tn2, EPS2))
            w = dot * rt * rs_ref[b, r0:r0 + RC, :]
            part += jnp.sum(w, axis=0, keepdims=True)
        prev = acc_ref[b, pl.ds(e, 1), :]
        acc_ref[b, pl.ds(e, 1), :] = jnp.where(s == 0, part, prev + part)

    @pl.when((s == NS - 1) & (e == E - 1))
    def _():
        for b in range(B):
            sims = acc_ref[b]
            idx_ref[b] = jnp.argmax(sims[:, 0], axis=0).astype(jnp.int32)


def _copy_kernel(idx_ref, t_ref, o_ref):
    del idx_ref
    o_ref[...] = t_ref[0]


@functools.partial(jax.jit, static_argnames=("interpret",))
def kernel(student_features, teacher_features, interpret=False):
    idx = pl.pallas_call(
        _sims_kernel,
        grid=(NS, E),
        in_specs=[
            pl.BlockSpec((B, S_BLK, D), lambda s, e: (0, s, 0)),
            pl.BlockSpec((1, B, S_BLK, D), lambda s, e: (e, 0, s, 0)),
        ],
        out_specs=pl.BlockSpec(memory_space=pltpu.SMEM),
        out_shape=jax.ShapeDtypeStruct((B,), jnp.int32),
        scratch_shapes=[
            pltpu.VMEM((B, E, 1), jnp.float32),
            pltpu.VMEM((B, S_BLK, 1), jnp.float32),
        ],
        compiler_params=pltpu.CompilerParams(
            dimension_semantics=("arbitrary", "arbitrary"),
        ),
        interpret=interpret,
    )(student_features, teacher_features)

    grid_spec = pltpu.PrefetchScalarGridSpec(
        num_scalar_prefetch=1,
        grid=(B, NCB),
        in_specs=[
            pl.BlockSpec((1, 1, C_BLK, D), lambda b, s, idx_ref: (idx_ref[b], b, s, 0)),
        ],
        out_specs=pl.BlockSpec((1, C_BLK, D), lambda b, s, idx_ref: (b, s, 0)),
    )
    out = pl.pallas_call(
        _copy_kernel,
        grid_spec=grid_spec,
        out_shape=jax.ShapeDtypeStruct((B, S, D), jnp.float32),
        interpret=interpret,
    )(idx, teacher_features)
    return out


# speedup vs baseline: 9.0763x; 1.0186x over previous
import functools

import jax
import jax.numpy as jnp
from jax import lax
from jax.experimental import pallas as pl
from jax.experimental.pallas import tpu as pltpu

B, S, D, E = 2, 2048, 1024, 8
S_BLK = 2048
NS = S // S_BLK
C_BLK = 1024
NCB = S // C_BLK
RC = 64
NRC = S_BLK // RC
NK = D // 128
EPS2 = 1e-24


def _sims_kernel(s_ref, t_ref, idx_ref, acc_ref, rs_ref):
    s = pl.program_id(0)
    e = pl.program_id(1)
    for b in range(B):
        @pl.when(e == 0)
        def _():
            for rc in range(NRC):
                r0 = rc * RC
                sn_acc = jnp.zeros((RC, 128), jnp.float32)
                for k in range(NK):
                    sfk = s_ref[b, r0:r0 + RC, k * 128:(k + 1) * 128]
                    sn_acc += sfk * sfk
                sn2 = jnp.sum(sn_acc, axis=1, keepdims=True)
                rs_ref[b, r0:r0 + RC, :] = lax.rsqrt(jnp.maximum(sn2, EPS2))

        part = jnp.zeros((1, 1), jnp.float32)
        for rc in range(NRC):
            r0 = rc * RC
            dot_acc = jnp.zeros((RC, 128), jnp.float32)
            tn_acc = jnp.zeros((RC, 128), jnp.float32)
            for k in range(NK):
                sfk = s_ref[b, r0:r0 + RC, k * 128:(k + 1) * 128]
                tfk = t_ref[0, b, r0:r0 + RC, k * 128:(k + 1) * 128]
                dot_acc += sfk * tfk
                tn_acc += tfk * tfk
            dot = jnp.sum(dot_acc, axis=1, keepdims=True)
            tn2 = jnp.sum(tn_acc, axis=1, keepdims=True)
            rt = lax.rsqrt(jnp.maximum(tn2, EPS2))
            w = dot * rt * rs_ref[b, r0:r0 + RC, :]
            part += jnp.sum(w, axis=0, keepdims=True)
        prev = acc_ref[b, pl.ds(e, 1), :]
        acc_ref[b, pl.ds(e, 1), :] = jnp.where(s == 0, part, prev + part)

    @pl.when((s == NS - 1) & (e == E - 1))
    def _():
        for b in range(B):
            sims = acc_ref[b]
            idx_ref[b] = jnp.argmax(sims[:, 0], axis=0).astype(jnp.int32)


def _copy_kernel(idx_ref, t_ref, o_ref):
    del idx_ref
    o_ref[...] = t_ref[0]


@functools.partial(jax.jit, static_argnames=("interpret",))
def kernel(student_features, teacher_features, interpret=False):
    idx = pl.pallas_call(
        _sims_kernel,
        grid=(NS, E),
        in_specs=[
            pl.BlockSpec((B, S_BLK, D), lambda s, e: (0, s, 0)),
            pl.BlockSpec((1, B, S_BLK, D), lambda s, e: (e, 0, s, 0)),
        ],
        out_specs=pl.BlockSpec(memory_space=pltpu.SMEM),
        out_shape=jax.ShapeDtypeStruct((B,), jnp.int32),
        scratch_shapes=[
            pltpu.VMEM((B, E, 1), jnp.float32),
            pltpu.VMEM((B, S_BLK, 1), jnp.float32),
        ],
        compiler_params=pltpu.CompilerParams(
            dimension_semantics=("arbitrary", "arbitrary"),
        ),
        interpret=interpret,
    )(student_features, teacher_features)

    grid_spec = pltpu.PrefetchScalarGridSpec(
        num_scalar_prefetch=1,
        grid=(B, NCB),
        in_specs=[
            pl.BlockSpec((1, 1, C_BLK, D), lambda b, s, idx_ref: (idx_ref[b], b, s, 0)),
        ],
        out_specs=pl.BlockSpec((1, C_BLK, D), lambda b, s, idx_ref: (b, s, 0)),
    )
    out = pl.pallas_call(
        _copy_kernel,
        grid_spec=grid_spec,
        out_shape=jax.ShapeDtypeStruct((B, S, D), jnp.float32),
        interpret=interpret,
    )(idx, teacher_features)
    return out


# speedup vs baseline: 9.2650x; 1.0208x over previous
import jax
import jax.numpy as jnp
from jax import lax
from jax.experimental import pallas as pl
from jax.experimental.pallas import tpu as pltpu

B, S, D, E = 2, 2048, 1024, 8
S_BLK = 2048
NS = S // S_BLK
C_BLK = 2048
NCB = S // C_BLK
RC = 64
NRC = S_BLK // RC
NK = D // 128
EPS2 = 1e-24


def _sims_kernel(s_ref, t_ref, idx_ref, acc_ref, rs_ref):
    s = pl.program_id(0)
    e = pl.program_id(1)
    for b in range(B):
        @pl.when(e == 0)
        def _():
            for rc in range(NRC):
                r0 = rc * RC
                sn_acc = jnp.zeros((RC, 128), jnp.float32)
                for k in range(NK):
                    sfk = s_ref[b, r0:r0 + RC, k * 128:(k + 1) * 128]
                    sn_acc += sfk * sfk
                sn2 = jnp.sum(sn_acc, axis=1, keepdims=True)
                rs_ref[b, r0:r0 + RC, :] = lax.rsqrt(jnp.maximum(sn2, EPS2))

        part = jnp.zeros((1, 1), jnp.float32)
        for rc in range(NRC):
            r0 = rc * RC
            dot_acc = jnp.zeros((RC, 128), jnp.float32)
            tn_acc = jnp.zeros((RC, 128), jnp.float32)
            for k in range(NK):
                sfk = s_ref[b, r0:r0 + RC, k * 128:(k + 1) * 128]
                tfk = t_ref[0, b, r0:r0 + RC, k * 128:(k + 1) * 128]
                dot_acc += sfk * tfk
                tn_acc += tfk * tfk
            dot = jnp.sum(dot_acc, axis=1, keepdims=True)
            tn2 = jnp.sum(tn_acc, axis=1, keepdims=True)
            rt = lax.rsqrt(jnp.maximum(tn2, EPS2))
            w = dot * rt * rs_ref[b, r0:r0 + RC, :]
            part += jnp.sum(w, axis=0, keepdims=True)
        prev = acc_ref[b, pl.ds(e, 1), :]
        acc_ref[b, pl.ds(e, 1), :] = jnp.where(s == 0, part, prev + part)

    @pl.when((s == NS - 1) & (e == E - 1))
    def _():
        for b in range(B):
            sims = acc_ref[b]
            idx_ref[b] = jnp.argmax(sims[:, 0], axis=0).astype(jnp.int32)


def _copy_kernel(idx_ref, t_ref, o_ref):
    del idx_ref
    o_ref[...] = t_ref[0]


@jax.jit
def kernel(student_features, teacher_features):
    idx = pl.pallas_call(
        _sims_kernel,
        grid=(NS, E),
        in_specs=[
            pl.BlockSpec((B, S_BLK, D), lambda s, e: (0, s, 0)),
            pl.BlockSpec((1, B, S_BLK, D), lambda s, e: (e, 0, s, 0)),
        ],
        out_specs=pl.BlockSpec(memory_space=pltpu.SMEM),
        out_shape=jax.ShapeDtypeStruct((B,), jnp.int32),
        scratch_shapes=[
            pltpu.VMEM((B, E, 1), jnp.float32),
            pltpu.VMEM((B, S_BLK, 1), jnp.float32),
        ],
        compiler_params=pltpu.CompilerParams(
            dimension_semantics=("arbitrary", "arbitrary"),
        ),
    )(student_features, teacher_features)

    grid_spec = pltpu.PrefetchScalarGridSpec(
        num_scalar_prefetch=1,
        grid=(B, NCB),
        in_specs=[
            pl.BlockSpec((1, 1, C_BLK, D), lambda b, s, idx_ref: (idx_ref[b], b, s, 0)),
        ],
        out_specs=pl.BlockSpec((1, C_BLK, D), lambda b, s, idx_ref: (b, s, 0)),
    )
    out = pl.pallas_call(
        _copy_kernel,
        grid_spec=grid_spec,
        out_shape=jax.ShapeDtypeStruct((B, S, D), jnp.float32),
    )(idx, teacher_features)
    return out
